# baseline (speedup 1.0000x reference)
import functools

import jax
import jax.numpy as jnp
from jax import lax
from jax.experimental import pallas as pl
from jax.experimental.pallas import tpu as pltpu
from jax.experimental.pallas import tpu_sc as plsc

_N = 10000
_E = 320000
_D = 128
_H = 64
_SB = 256
_NSLOT = 4
_NIDX = 8
_NT = 16
_NP = 10240
_RPT = _NP // _NT
_EP = 327680
_NSB = _EP // _SB
_ROWBLK = 640


def _sc_mesh():
    return plsc.VectorSubcoreMesh(core_axis_name="c", subcore_axis_name="s")


def _sc_scratch():
    sc = []
    for _ in range(_NIDX):
        sc += [pltpu.VMEM((_SB,), jnp.int32),
               pltpu.VMEM((_SB,), jnp.int32),
               pltpu.VMEM((_SB,), jnp.float32)]
    sc += [pltpu.VMEM((_SB, _H), jnp.float32)] * _NSLOT
    sc += [pltpu.VMEM_SHARED((_NP, _H), jnp.float32)]
    sc += [pltpu.SemaphoreType.DMA] * (_NIDX + 2 * _NSLOT)
    return sc


def _unpack_scratch(scr):
    idxb = [tuple(scr[3 * i:3 * i + 3]) for i in range(_NIDX)]
    base = 3 * _NIDX
    rows = list(scr[base:base + _NSLOT])
    acc_sh = scr[base + _NSLOT]
    sems = scr[base + _NSLOT + 1:]
    sid = sems[:_NIDX]
    sg = sems[_NIDX:_NIDX + _NSLOT]
    ss = sems[_NIDX + _NSLOT:]
    return idxb, rows, acc_sh, sid, sg, ss


def _stripe_pieces():
    pieces, off = [], 0
    while off < _RPT:
        ln = min(_SB, _RPT - off)
        pieces.append((off, ln))
        off += ln
    return pieces


def _zero_acc(acc_sh, rows0, s):
    zero16 = jnp.zeros((16,), jnp.float32)

    def zrow(r, carry):
        for j in range(_H // 16):
            rows0[r, pl.ds(j * 16, 16)] = zero16
        return carry

    lax.fori_loop(0, _SB, zrow, 0)
    r0 = s * _RPT
    for off, ln in _stripe_pieces():
        pltpu.sync_copy(rows0.at[pl.ds(0, ln)],
                        acc_sh.at[pl.ds(r0 + off, ln)])


def _write_out(rows0, acc_sh, o_slice, s):
    r0 = s * _RPT
    for off, ln in _stripe_pieces():
        pltpu.sync_copy(acc_sh.at[pl.ds(r0 + off, ln)],
                        rows0.at[pl.ds(0, ln)])
        pltpu.sync_copy(rows0.at[pl.ds(0, ln)],
                        o_slice.at[pl.ds(r0 + off, ln)])


def _pipeline(sb0, nsb, scr, tbl, src_h, dst_h, w_h):
    idxb, rows, acc_sh, sid, sg, ss = scr

    def load_idx(i, sbi):
        src_v, dst_v, w_v = idxb[i]
        e0 = (sb0 + sbi) * _SB
        pltpu.async_copy(src_h.at[pl.ds(e0, _SB)], src_v, sid[i])
        pltpu.async_copy(dst_h.at[pl.ds(e0, _SB)], dst_v, sid[i])
        pltpu.async_copy(w_h.at[pl.ds(e0, _SB)], w_v, sid[i])

    def wait_idx(i):
        src_v, dst_v, w_v = idxb[i]
        e0 = sb0 * _SB
        pltpu.make_async_copy(src_h.at[pl.ds(e0, _SB)], src_v, sid[i]).wait()
        pltpu.make_async_copy(dst_h.at[pl.ds(e0, _SB)], dst_v, sid[i]).wait()
        pltpu.make_async_copy(w_h.at[pl.ds(e0, _SB)], w_v, sid[i]).wait()

    def fire_gather(x, i):
        pltpu.async_copy(tbl.at[idxb[i][0]], rows[x], sg[x])

    def drain_gather(x, i):
        pltpu.make_async_copy(tbl.at[idxb[i][0]], rows[x], sg[x]).wait()

    def fire_scatter(x, i):
        pltpu.async_copy(rows[x], acc_sh.at[idxb[i][1]], ss[x], add=True)

    def drain_scatter(x, i):
        pltpu.make_async_copy(rows[x], acc_sh.at[idxb[i][1]], ss[x]).wait()

    def scale(x, i):
        w_v = idxb[i][2]
        rows_v = rows[x]

        def grp(g, carry):
            wv16 = w_v[pl.ds(g * 16, 16)]
            for ii in range(16):
                r = g * 16 + ii
                wv = wv16[ii]
                for q in range(_H // 16):
                    sl = pl.ds(q * 16, 16)
                    rows_v[r, sl] = rows_v[r, sl] * wv
            return carry

        lax.fori_loop(0, _SB // 16, grp, 0)

    for kk in range(_NSLOT):
        load_idx(kk, kk)
    for kk in range(_NSLOT):
        wait_idx(kk)
        fire_gather(kk, kk)

    def octet(qi, carry):
        for j in range(_NIDX):
            k = qi * _NIDX + j
            x = j % _NSLOT
            ip = (j + _NSLOT) % _NIDX

            @pl.when(k + _NSLOT < nsb)
            def _(ip=ip, k=k):
                load_idx(ip, k + _NSLOT)

            z = (j - 1) % _NSLOT
            iz_old = (j - 1) % _NIDX
            iz_new = (j + 3) % _NIDX

            @pl.when(jnp.logical_and(k >= 1, k + 3 < nsb))
            def _(z=z, iz_old=iz_old, iz_new=iz_new, k=k):
                drain_scatter(z, iz_old)
                wait_idx(iz_new)
                fire_gather(z, iz_new)

            drain_gather(x, j)
            scale(x, j)
            fire_scatter(x, j)
        return carry

    lax.fori_loop(0, nsb // _NIDX, octet, 0)

    for kk in range(_NSLOT):
        j = (nsb - _NSLOT + kk) % _NSLOT
        drain_scatter(j, (nsb - _NSLOT + kk) % _NIDX)


def _spmm_hop1(src1, dst1, w1, t0, t1):
    spt = _NSB // _NT

    @functools.partial(
        pl.kernel,
        mesh=_sc_mesh(),
        out_type=[jax.ShapeDtypeStruct((_NP, _H), jnp.float32),
                  jax.ShapeDtypeStruct((_NP, _H), jnp.float32)],
        scratch_types=_sc_scratch(),
        compiler_params=pltpu.CompilerParams(use_tc_tiling_on_sc=False),
    )
    def k(src_h, dst_h, w_h, t0_h, t1_h, o0_h, o1_h, *scratch):
        c = lax.axis_index("c")
        s = lax.axis_index("s")
        scr = _unpack_scratch(scratch)
        _zero_acc(scr[2], scr[1][0], s)
        plsc.subcore_barrier()

        sb0 = s * spt

        @pl.when(c == 0)
        def _():
            _pipeline(sb0, spt, scr, t0_h, src_h, dst_h, w_h)

        @pl.when(c == 1)
        def _():
            _pipeline(sb0, spt, scr, t1_h, src_h, dst_h, w_h)

        plsc.subcore_barrier()

        @pl.when(c == 0)
        def _():
            _write_out(scr[1][0], scr[2], o0_h, s)

        @pl.when(c == 1)
        def _():
            _write_out(scr[1][0], scr[2], o1_h, s)

    return k(src1, dst1, w1, t0, t1)


def _spmm_hop2(src1, dst1, w1, t):
    half_sb = _NSB // 2
    spt = half_sb // _NT

    @functools.partial(
        pl.kernel,
        mesh=_sc_mesh(),
        out_type=jax.ShapeDtypeStruct((2, _NP, _H), jnp.float32),
        scratch_types=_sc_scratch(),
        compiler_params=pltpu.CompilerParams(use_tc_tiling_on_sc=False),
    )
    def k(src_h, dst_h, w_h, t_h, o_h, *scratch):
        c = lax.axis_index("c")
        s = lax.axis_index("s")
        scr = _unpack_scratch(scratch)
        _zero_acc(scr[2], scr[1][0], s)
        plsc.subcore_barrier()

        sb0 = c * half_sb + s * spt
        _pipeline(sb0, spt, scr, t_h, src_h, dst_h, w_h)

        plsc.subcore_barrier()
        _write_out(scr[1][0], scr[2], o_h.at[c], s)

    return k(src1, dst1, w1, t)


def _tc_first(x, wcat):
    def body(x_ref, w_ref, o0_ref, o1_ref):
        u = jnp.dot(x_ref[...], w_ref[...],
                    preferred_element_type=jnp.float32)
        o0_ref[...] = u[:, :_H]
        o1_ref[...] = u[:, _H:]

    return pl.pallas_call(
        body,
        grid=(_NP // _ROWBLK,),
        in_specs=[pl.BlockSpec((_ROWBLK, _D), lambda i: (i, 0)),
                  pl.BlockSpec((_D, _D), lambda i: (0, 0))],
        out_specs=[pl.BlockSpec((_ROWBLK, _H), lambda i: (i, 0)),
                   pl.BlockSpec((_ROWBLK, _H), lambda i: (i, 0))],
        out_shape=[jax.ShapeDtypeStruct((_NP, _H), jnp.float32),
                   jax.ShapeDtypeStruct((_NP, _H), jnp.float32)],
    )(x, wcat)


def _tc_layer(keep, p0, p1, wcat, bvec):
    def body(k_ref, p0_ref, p1_ref, w_ref, b_ref, o0_ref, o1_ref):
        wl = w_ref[...]
        upper = p0_ref[...] + p1_ref[...]
        u = (jnp.dot(k_ref[...], wl[:_H, :],
                     preferred_element_type=jnp.float32)
             + jnp.dot(upper, wl[_H:, :],
                       preferred_element_type=jnp.float32)
             + jnp.dot(b_ref[...], wl, preferred_element_type=jnp.float32))
        o0_ref[...] = u[:, :_H]
        o1_ref[...] = u[:, _H:]

    return pl.pallas_call(
        body,
        grid=(_NP // _ROWBLK,),
        in_specs=[pl.BlockSpec((_ROWBLK, _H), lambda i: (i, 0)),
                  pl.BlockSpec((_ROWBLK, _H), lambda i: (i, 0)),
                  pl.BlockSpec((_ROWBLK, _H), lambda i: (i, 0)),
                  pl.BlockSpec((_D, _D), lambda i: (0, 0)),
                  pl.BlockSpec((1, _D), lambda i: (0, 0))],
        out_specs=[pl.BlockSpec((_ROWBLK, _H), lambda i: (i, 0)),
                   pl.BlockSpec((_ROWBLK, _H), lambda i: (i, 0))],
        out_shape=[jax.ShapeDtypeStruct((_NP, _H), jnp.float32),
                   jax.ShapeDtypeStruct((_NP, _H), jnp.float32)],
    )(keep, p0, p1, wcat, bvec)


def _tc_final(keep, p0, p1, wcp, bvec, bcp):
    def body(k_ref, p0_ref, p1_ref, w_ref, b_ref, bc_ref, o_ref):
        wl = w_ref[...]
        upper = p0_ref[...] + p1_ref[...]
        o_ref[...] = (jnp.dot(k_ref[...], wl[:_H, :],
                              preferred_element_type=jnp.float32)
                      + jnp.dot(upper, wl[_H:, :],
                                preferred_element_type=jnp.float32)
                      + jnp.dot(b_ref[...], wl,
                                preferred_element_type=jnp.float32)
                      + bc_ref[...])

    return pl.pallas_call(
        body,
        grid=(_NP // _ROWBLK,),
        in_specs=[pl.BlockSpec((_ROWBLK, _H), lambda i: (i, 0)),
                  pl.BlockSpec((_ROWBLK, _H), lambda i: (i, 0)),
                  pl.BlockSpec((_ROWBLK, _H), lambda i: (i, 0)),
                  pl.BlockSpec((_D, _D), lambda i: (0, 0)),
                  pl.BlockSpec((1, _D), lambda i: (0, 0)),
                  pl.BlockSpec((1, _D), lambda i: (0, 0))],
        out_specs=pl.BlockSpec((_ROWBLK, _D), lambda i: (i, 0)),
        out_shape=jax.ShapeDtypeStruct((_NP, _D), jnp.float32),
    )(keep, p0, p1, wcp, bvec, bcp)


def kernel(x, edge_index, edge_weight, W, b, Wc, bc):
    pad = _EP - _E
    src1 = jnp.concatenate([edge_index[0], jnp.zeros((pad,), jnp.int32)])
    dst1 = jnp.concatenate([edge_index[1], jnp.zeros((pad,), jnp.int32)])
    w1 = jnp.concatenate([edge_weight, jnp.zeros((pad,), jnp.float32)])
    xp = jnp.pad(x, ((0, _NP - _N), (0, 0)))
    nclass = Wc.shape[1]

    t0, t1 = _tc_first(xp, jnp.concatenate([W[0, 0], W[0, 1]], axis=1))
    for l in range(W.shape[0]):
        keep, upper = _spmm_hop1(src1, dst1, w1, t0, t1)
        parts = _spmm_hop2(src1, dst1, w1, upper)
        p0, p1 = parts[0], parts[1]
        bvec = jnp.concatenate([b[l, 0], b[l, 1]])[None, :]
        if l + 1 < W.shape[0]:
            wcat = jnp.concatenate([W[l + 1, 0], W[l + 1, 1]], axis=1)
            t0, t1 = _tc_layer(keep, p0, p1, wcat, bvec)
        else:
            wcp = jnp.pad(Wc, ((0, 0), (0, _D - nclass)))
            bcp = jnp.pad(bc, (0, _D - nclass))[None, :]
            out = _tc_final(keep, p0, p1, wcp, bvec, bcp)
            return out[:_N, :nclass]


# speedup vs baseline: 1.1313x; 1.1313x over previous
import functools

import jax
import jax.numpy as jnp
from jax import lax
from jax.experimental import pallas as pl
from jax.experimental.pallas import tpu as pltpu
from jax.experimental.pallas import tpu_sc as plsc

_N = 10000
_E = 320000
_EP = 327680
_D = 128
_H = 64
_CH = 128
_SBC = 4
_SB = _SBC * _CH
_NT = 16
_NP = 10240
_RPT = _NP // _NT
_NCHUNK = _EP // _CH
_ROWBLK = 640


def _sc_mesh():
    return plsc.VectorSubcoreMesh(core_axis_name="c", subcore_axis_name="s")


def _sc_scratch():
    bufs = []
    for _ in range(2):
        bufs += [pltpu.VMEM((_SBC, _CH), jnp.int32),
                 pltpu.VMEM((_SBC, _CH), jnp.int32),
                 pltpu.VMEM((_SBC, _CH), jnp.float32),
                 pltpu.VMEM((_SB, _H), jnp.float32)]
    return bufs + [
        pltpu.VMEM_SHARED((_NP, _H), jnp.float32),
        pltpu.SemaphoreType.DMA,
        pltpu.SemaphoreType.DMA,
        pltpu.SemaphoreType.DMA,
        pltpu.SemaphoreType.DMA,
        pltpu.SemaphoreType.DMA,
        pltpu.SemaphoreType.DMA,
    ]


def _pipeline(slab0, nsb, bufs, acc_sh, t_h, src2_h, dst2_h, w2_h):
    def load_idx(hb, sbi):
        src_v, dst_v, w_v, sem = hb[0], hb[1], hb[2], hb[4]
        row = slab0 + sbi * _SBC
        pltpu.async_copy(src2_h.at[pl.ds(row, _SBC)], src_v, sem)
        pltpu.async_copy(dst2_h.at[pl.ds(row, _SBC)], dst_v, sem)
        pltpu.async_copy(w2_h.at[pl.ds(row, _SBC)], w_v, sem)

    def wait_idx(hb):
        src_v, dst_v, w_v, sem = hb[0], hb[1], hb[2], hb[4]
        row = slab0
        pltpu.make_async_copy(src2_h.at[pl.ds(row, _SBC)], src_v, sem).wait()
        pltpu.make_async_copy(dst2_h.at[pl.ds(row, _SBC)], dst_v, sem).wait()
        pltpu.make_async_copy(w2_h.at[pl.ds(row, _SBC)], w_v, sem).wait()

    def fire_gathers(hb):
        src_v, rows_v, sem = hb[0], hb[3], hb[5]
        for j in range(_SBC):
            pltpu.async_copy(t_h.at[src_v.at[j]],
                             rows_v.at[pl.ds(j * _CH, _CH)], sem)

    def drain_gathers(hb):
        src_v, rows_v, sem = hb[0], hb[3], hb[5]
        for j in range(_SBC):
            pltpu.make_async_copy(t_h.at[src_v.at[j]],
                                  rows_v.at[pl.ds(j * _CH, _CH)], sem).wait()

    def scale_scatter(hb):
        dst_v, w_v, rows_v, sem = hb[1], hb[2], hb[3], hb[6]
        for j in range(_SBC):

            def grp(g, carry, j=j):
                wv16 = w_v[j, pl.ds(g * 16, 16)]
                for i in range(16):
                    r = j * _CH + g * 16 + i
                    wv = wv16[i]
                    for q in range(_H // 16):
                        sl = pl.ds(q * 16, 16)
                        rows_v[r, sl] = rows_v[r, sl] * wv
                return carry

            lax.fori_loop(0, _CH // 16, grp, 0)
            pltpu.async_copy(rows_v.at[pl.ds(j * _CH, _CH)],
                             acc_sh.at[dst_v.at[j]], sem, add=True)

    def drain_scatters(hb):
        dst_v, rows_v, sem = hb[1], hb[3], hb[6]
        for j in range(_SBC):
            pltpu.make_async_copy(rows_v.at[pl.ds(j * _CH, _CH)],
                                  acc_sh.at[dst_v.at[j]], sem).wait()

    buf_a, buf_b = bufs
    npairs = nsb // 2

    load_idx(buf_a, 0)
    load_idx(buf_b, 1)
    wait_idx(buf_a)
    fire_gathers(buf_a)
    wait_idx(buf_b)
    fire_gathers(buf_b)

    def half(hb, sb_next, is_not_last):
        drain_gathers(hb)
        scale_scatter(hb)
        drain_scatters(hb)

        @pl.when(is_not_last)
        def _():
            load_idx(hb, sb_next)
            wait_idx(hb)
            fire_gathers(hb)

    def pair(pi, carry):
        not_last = pi < npairs - 1
        half(buf_a, pi * 2 + 2, not_last)
        half(buf_b, pi * 2 + 3, not_last)
        return carry

    lax.fori_loop(0, npairs, pair, 0)


def _stripe_pieces():
    pieces, off = [], 0
    while off < _RPT:
        ln = min(_SB, _RPT - off)
        pieces.append((off, ln))
        off += ln
    return pieces


def _zero_acc(acc_sh, rows_a, s):
    zero16 = jnp.zeros((16,), jnp.float32)

    def zrow(r, carry):
        for j in range(_H // 16):
            rows_a[r, pl.ds(j * 16, 16)] = zero16
        return carry

    lax.fori_loop(0, _SB, zrow, 0)
    r0 = s * _RPT
    for off, ln in _stripe_pieces():
        pltpu.sync_copy(rows_a.at[pl.ds(0, ln)],
                        acc_sh.at[pl.ds(r0 + off, ln)])


def _write_out(rows_a, acc_sh, o_slice, s):
    r0 = s * _RPT
    for off, ln in _stripe_pieces():
        pltpu.sync_copy(acc_sh.at[pl.ds(r0 + off, ln)],
                        rows_a.at[pl.ds(0, ln)])
        pltpu.sync_copy(rows_a.at[pl.ds(0, ln)],
                        o_slice.at[pl.ds(r0 + off, ln)])


def _spmm_hop1(src2, dst2, w2, t0, t1):
    cpt = _NCHUNK // _NT
    nsb = cpt // _SBC

    @functools.partial(
        pl.kernel,
        mesh=_sc_mesh(),
        out_type=[jax.ShapeDtypeStruct((_NP, _H), jnp.float32),
                  jax.ShapeDtypeStruct((_NP, _H), jnp.float32)],
        scratch_types=_sc_scratch(),
        compiler_params=pltpu.CompilerParams(use_tc_tiling_on_sc=False),
    )
    def k(src2_h, dst2_h, w2_h, t0_h, t1_h, o0_h, o1_h,
          src_a, dst_a, w_a, rows_a, src_b, dst_b, w_b, rows_b, acc_sh,
          sida, sidb, sga, sgb, ssa, ssb):
        c = lax.axis_index("c")
        s = lax.axis_index("s")
        buf_a = (src_a, dst_a, w_a, rows_a, sida, sga, ssa)
        buf_b = (src_b, dst_b, w_b, rows_b, sidb, sgb, ssb)
        _zero_acc(acc_sh, rows_a, s)
        plsc.subcore_barrier()

        slab0 = s * cpt

        @pl.when(c == 0)
        def _():
            _pipeline(slab0, nsb, (buf_a, buf_b), acc_sh, t0_h,
                      src2_h, dst2_h, w2_h)

        @pl.when(c == 1)
        def _():
            _pipeline(slab0, nsb, (buf_a, buf_b), acc_sh, t1_h,
                      src2_h, dst2_h, w2_h)

        plsc.subcore_barrier()

        @pl.when(c == 0)
        def _():
            _write_out(rows_a, acc_sh, o0_h, s)

        @pl.when(c == 1)
        def _():
            _write_out(rows_a, acc_sh, o1_h, s)

    return k(src2, dst2, w2, t0, t1)


def _spmm_hop2(src2, dst2, w2, t):
    half = _NCHUNK // 2
    cpt = half // _NT
    nsb = cpt // _SBC

    @functools.partial(
        pl.kernel,
        mesh=_sc_mesh(),
        out_type=jax.ShapeDtypeStruct((2, _NP, _H), jnp.float32),
        scratch_types=_sc_scratch(),
        compiler_params=pltpu.CompilerParams(use_tc_tiling_on_sc=False),
    )
    def k(src2_h, dst2_h, w2_h, t_h, o_h,
          src_a, dst_a, w_a, rows_a, src_b, dst_b, w_b, rows_b, acc_sh,
          sida, sidb, sga, sgb, ssa, ssb):
        c = lax.axis_index("c")
        s = lax.axis_index("s")
        buf_a = (src_a, dst_a, w_a, rows_a, sida, sga, ssa)
        buf_b = (src_b, dst_b, w_b, rows_b, sidb, sgb, ssb)
        _zero_acc(acc_sh, rows_a, s)
        plsc.subcore_barrier()

        slab0 = c * half + s * cpt
        _pipeline(slab0, nsb, (buf_a, buf_b), acc_sh, t_h,
                  src2_h, dst2_h, w2_h)

        plsc.subcore_barrier()
        _write_out(rows_a, acc_sh, o_h.at[c], s)

    return k(src2, dst2, w2, t)


def _tc_first(x, wcat):
    def body(x_ref, w_ref, o0_ref, o1_ref):
        u = jnp.dot(x_ref[...], w_ref[...],
                    preferred_element_type=jnp.float32)
        o0_ref[...] = u[:, :_H]
        o1_ref[...] = u[:, _H:]

    return pl.pallas_call(
        body,
        grid=(_NP // _ROWBLK,),
        in_specs=[pl.BlockSpec((_ROWBLK, _D), lambda i: (i, 0)),
                  pl.BlockSpec((_D, _D), lambda i: (0, 0))],
        out_specs=[pl.BlockSpec((_ROWBLK, _H), lambda i: (i, 0)),
                   pl.BlockSpec((_ROWBLK, _H), lambda i: (i, 0))],
        out_shape=[jax.ShapeDtypeStruct((_NP, _H), jnp.float32),
                   jax.ShapeDtypeStruct((_NP, _H), jnp.float32)],
    )(x, wcat)


def _tc_layer(keep, p0, p1, wcat, bvec):
    def body(k_ref, p0_ref, p1_ref, w_ref, b_ref, o0_ref, o1_ref):
        wl = w_ref[...]
        upper = p0_ref[...] + p1_ref[...]
        u = (jnp.dot(k_ref[...], wl[:_H, :],
                     preferred_element_type=jnp.float32)
             + jnp.dot(upper, wl[_H:, :],
                       preferred_element_type=jnp.float32)
             + jnp.dot(b_ref[...], wl, preferred_element_type=jnp.float32))
        o0_ref[...] = u[:, :_H]
        o1_ref[...] = u[:, _H:]

    return pl.pallas_call(
        body,
        grid=(_NP // _ROWBLK,),
        in_specs=[pl.BlockSpec((_ROWBLK, _H), lambda i: (i, 0)),
                  pl.BlockSpec((_ROWBLK, _H), lambda i: (i, 0)),
                  pl.BlockSpec((_ROWBLK, _H), lambda i: (i, 0)),
                  pl.BlockSpec((_D, _D), lambda i: (0, 0)),
                  pl.BlockSpec((1, _D), lambda i: (0, 0))],
        out_specs=[pl.BlockSpec((_ROWBLK, _H), lambda i: (i, 0)),
                   pl.BlockSpec((_ROWBLK, _H), lambda i: (i, 0))],
        out_shape=[jax.ShapeDtypeStruct((_NP, _H), jnp.float32),
                   jax.ShapeDtypeStruct((_NP, _H), jnp.float32)],
    )(keep, p0, p1, wcat, bvec)


def _tc_final(keep, p0, p1, wcp, bvec, bcp):
    def body(k_ref, p0_ref, p1_ref, w_ref, b_ref, bc_ref, o_ref):
        wl = w_ref[...]
        upper = p0_ref[...] + p1_ref[...]
        o_ref[...] = (jnp.dot(k_ref[...], wl[:_H, :],
                              preferred_element_type=jnp.float32)
                      + jnp.dot(upper, wl[_H:, :],
                                preferred_element_type=jnp.float32)
                      + jnp.dot(b_ref[...], wl,
                                preferred_element_type=jnp.float32)
                      + bc_ref[...])

    return pl.pallas_call(
        body,
        grid=(_NP // _ROWBLK,),
        in_specs=[pl.BlockSpec((_ROWBLK, _H), lambda i: (i, 0)),
                  pl.BlockSpec((_ROWBLK, _H), lambda i: (i, 0)),
                  pl.BlockSpec((_ROWBLK, _H), lambda i: (i, 0)),
                  pl.BlockSpec((_D, _D), lambda i: (0, 0)),
                  pl.BlockSpec((1, _D), lambda i: (0, 0)),
                  pl.BlockSpec((1, _D), lambda i: (0, 0))],
        out_specs=pl.BlockSpec((_ROWBLK, _D), lambda i: (i, 0)),
        out_shape=jax.ShapeDtypeStruct((_NP, _D), jnp.float32),
    )(keep, p0, p1, wcp, bvec, bcp)


def kernel(x, edge_index, edge_weight, W, b, Wc, bc):
    pad = _EP - _E
    src2 = jnp.concatenate(
        [edge_index[0], jnp.zeros((pad,), jnp.int32)]).reshape(_NCHUNK, _CH)
    dst2 = jnp.concatenate(
        [edge_index[1], jnp.zeros((pad,), jnp.int32)]).reshape(_NCHUNK, _CH)
    w2 = jnp.concatenate(
        [edge_weight, jnp.zeros((pad,), jnp.float32)]).reshape(_NCHUNK, _CH)
    xp = jnp.pad(x, ((0, _NP - _N), (0, 0)))
    nclass = Wc.shape[1]

    t0, t1 = _tc_first(xp, jnp.concatenate([W[0, 0], W[0, 1]], axis=1))
    for l in range(W.shape[0]):
        keep, upper = _spmm_hop1(src2, dst2, w2, t0, t1)
        parts = _spmm_hop2(src2, dst2, w2, upper)
        p0, p1 = parts[0], parts[1]
        bvec = jnp.concatenate([b[l, 0], b[l, 1]])[None, :]
        if l + 1 < W.shape[0]:
            wcat = jnp.concatenate([W[l + 1, 0], W[l + 1, 1]], axis=1)
            t0, t1 = _tc_layer(keep, p0, p1, wcat, bvec)
        else:
            wcp = jnp.pad(Wc, ((0, 0), (0, _D - nclass)))
            bcp = jnp.pad(bc, (0, _D - nclass))[None, :]
            out = _tc_final(keep, p0, p1, wcp, bvec, bcp)
            return out[:_N, :nclass]


# speedup vs baseline: 1.1391x; 1.0069x over previous
import functools

import jax
import jax.numpy as jnp
from jax import lax
from jax.experimental import pallas as pl
from jax.experimental.pallas import tpu as pltpu
from jax.experimental.pallas import tpu_sc as plsc

_N = 10000
_E = 320000
_EP = 327680
_D = 128
_H = 64
_CH = 128
_SBC = 4
_SB = _SBC * _CH
_NT = 16
_NP = 10240
_RPT = _NP // _NT
_NCHUNK = _EP // _CH
_ROWBLK = 1024


def _sc_mesh():
    return plsc.VectorSubcoreMesh(core_axis_name="c", subcore_axis_name="s")


def _sc_scratch():
    bufs = []
    for _ in range(2):
        bufs += [pltpu.VMEM((_SBC, _CH), jnp.int32),
                 pltpu.VMEM((_SBC, _CH), jnp.int32),
                 pltpu.VMEM((_SBC, _CH), jnp.float32),
                 pltpu.VMEM((_SB, _H), jnp.float32)]
    return bufs + [
        pltpu.VMEM_SHARED((_NP, _H), jnp.float32),
        pltpu.SemaphoreType.DMA,
        pltpu.SemaphoreType.DMA,
        pltpu.SemaphoreType.DMA,
        pltpu.SemaphoreType.DMA,
        pltpu.SemaphoreType.DMA,
        pltpu.SemaphoreType.DMA,
    ]


def _pipeline(slab0, nsb, bufs, acc_sh, t_h, src2_h, dst2_h, w2_h):
    def load_idx(hb, sbi):
        src_v, dst_v, w_v, sem = hb[0], hb[1], hb[2], hb[4]
        row = slab0 + sbi * _SBC
        pltpu.async_copy(src2_h.at[pl.ds(row, _SBC)], src_v, sem)
        pltpu.async_copy(dst2_h.at[pl.ds(row, _SBC)], dst_v, sem)
        pltpu.async_copy(w2_h.at[pl.ds(row, _SBC)], w_v, sem)

    def wait_idx(hb):
        src_v, dst_v, w_v, sem = hb[0], hb[1], hb[2], hb[4]
        row = slab0
        pltpu.make_async_copy(src2_h.at[pl.ds(row, _SBC)], src_v, sem).wait()
        pltpu.make_async_copy(dst2_h.at[pl.ds(row, _SBC)], dst_v, sem).wait()
        pltpu.make_async_copy(w2_h.at[pl.ds(row, _SBC)], w_v, sem).wait()

    def fire_gathers(hb):
        src_v, rows_v, sem = hb[0], hb[3], hb[5]
        for j in range(_SBC):
            pltpu.async_copy(t_h.at[src_v.at[j]],
                             rows_v.at[pl.ds(j * _CH, _CH)], sem)

    def drain_gathers(hb):
        src_v, rows_v, sem = hb[0], hb[3], hb[5]
        for j in range(_SBC):
            pltpu.make_async_copy(t_h.at[src_v.at[j]],
                                  rows_v.at[pl.ds(j * _CH, _CH)], sem).wait()

    def scale_scatter(hb):
        dst_v, w_v, rows_v, sem = hb[1], hb[2], hb[3], hb[6]
        for j in range(_SBC):

            def grp(g, carry, j=j):
                wv16 = w_v[j, pl.ds(g * 16, 16)]
                for i in range(16):
                    r = j * _CH + g * 16 + i
                    wv = wv16[i]
                    for q in range(_H // 16):
                        sl = pl.ds(q * 16, 16)
                        rows_v[r, sl] = rows_v[r, sl] * wv
                return carry

            lax.fori_loop(0, _CH // 16, grp, 0)
            pltpu.async_copy(rows_v.at[pl.ds(j * _CH, _CH)],
                             acc_sh.at[dst_v.at[j]], sem, add=True)

    def drain_scatters(hb):
        dst_v, rows_v, sem = hb[1], hb[3], hb[6]
        for j in range(_SBC):
            pltpu.make_async_copy(rows_v.at[pl.ds(j * _CH, _CH)],
                                  acc_sh.at[dst_v.at[j]], sem).wait()

    buf_a, buf_b = bufs
    npairs = nsb // 2

    load_idx(buf_a, 0)
    load_idx(buf_b, 1)
    wait_idx(buf_a)
    fire_gathers(buf_a)
    wait_idx(buf_b)
    fire_gathers(buf_b)

    def half(hb, sb_next, is_not_last):
        drain_gathers(hb)
        scale_scatter(hb)
        drain_scatters(hb)

        @pl.when(is_not_last)
        def _():
            load_idx(hb, sb_next)
            wait_idx(hb)
            fire_gathers(hb)

    def pair(pi, carry):
        not_last = pi < npairs - 1
        half(buf_a, pi * 2 + 2, not_last)
        half(buf_b, pi * 2 + 3, not_last)
        return carry

    lax.fori_loop(0, npairs, pair, 0)


def _stripe_pieces():
    pieces, off = [], 0
    while off < _RPT:
        ln = min(_SB, _RPT - off)
        pieces.append((off, ln))
        off += ln
    return pieces


def _zero_acc(acc_sh, rows_a, s):
    zero16 = jnp.zeros((16,), jnp.float32)

    def zrow(r, carry):
        for j in range(_H // 16):
            rows_a[r, pl.ds(j * 16, 16)] = zero16
        return carry

    lax.fori_loop(0, _SB, zrow, 0)
    r0 = s * _RPT
    for off, ln in _stripe_pieces():
        pltpu.sync_copy(rows_a.at[pl.ds(0, ln)],
                        acc_sh.at[pl.ds(r0 + off, ln)])


def _write_out(rows_a, acc_sh, o_slice, s):
    r0 = s * _RPT
    for off, ln in _stripe_pieces():
        pltpu.sync_copy(acc_sh.at[pl.ds(r0 + off, ln)],
                        rows_a.at[pl.ds(0, ln)])
        pltpu.sync_copy(rows_a.at[pl.ds(0, ln)],
                        o_slice.at[pl.ds(r0 + off, ln)])


def _spmm_hop1(src2, dst2, w2, t0, t1):
    cpt = _NCHUNK // _NT
    nsb = cpt // _SBC

    @functools.partial(
        pl.kernel,
        mesh=_sc_mesh(),
        out_type=[jax.ShapeDtypeStruct((_NP, _H), jnp.float32),
                  jax.ShapeDtypeStruct((_NP, _H), jnp.float32)],
        scratch_types=_sc_scratch(),
        compiler_params=pltpu.CompilerParams(use_tc_tiling_on_sc=False),
    )
    def k(src2_h, dst2_h, w2_h, t0_h, t1_h, o0_h, o1_h,
          src_a, dst_a, w_a, rows_a, src_b, dst_b, w_b, rows_b, acc_sh,
          sida, sidb, sga, sgb, ssa, ssb):
        c = lax.axis_index("c")
        s = lax.axis_index("s")
        buf_a = (src_a, dst_a, w_a, rows_a, sida, sga, ssa)
        buf_b = (src_b, dst_b, w_b, rows_b, sidb, sgb, ssb)
        _zero_acc(acc_sh, rows_a, s)
        plsc.subcore_barrier()

        slab0 = s * cpt

        @pl.when(c == 0)
        def _():
            _pipeline(slab0, nsb, (buf_a, buf_b), acc_sh, t0_h,
                      src2_h, dst2_h, w2_h)

        @pl.when(c == 1)
        def _():
            _pipeline(slab0, nsb, (buf_a, buf_b), acc_sh, t1_h,
                      src2_h, dst2_h, w2_h)

        plsc.subcore_barrier()

        @pl.when(c == 0)
        def _():
            _write_out(rows_a, acc_sh, o0_h, s)

        @pl.when(c == 1)
        def _():
            _write_out(rows_a, acc_sh, o1_h, s)

    return k(src2, dst2, w2, t0, t1)


def _spmm_hop2(src2, dst2, w2, t):
    half = _NCHUNK // 2
    cpt = half // _NT
    nsb = cpt // _SBC

    @functools.partial(
        pl.kernel,
        mesh=_sc_mesh(),
        out_type=jax.ShapeDtypeStruct((2, _NP, _H), jnp.float32),
        scratch_types=_sc_scratch(),
        compiler_params=pltpu.CompilerParams(use_tc_tiling_on_sc=False),
    )
    def k(src2_h, dst2_h, w2_h, t_h, o_h,
          src_a, dst_a, w_a, rows_a, src_b, dst_b, w_b, rows_b, acc_sh,
          sida, sidb, sga, sgb, ssa, ssb):
        c = lax.axis_index("c")
        s = lax.axis_index("s")
        buf_a = (src_a, dst_a, w_a, rows_a, sida, sga, ssa)
        buf_b = (src_b, dst_b, w_b, rows_b, sidb, sgb, ssb)
        _zero_acc(acc_sh, rows_a, s)
        plsc.subcore_barrier()

        slab0 = c * half + s * cpt
        _pipeline(slab0, nsb, (buf_a, buf_b), acc_sh, t_h,
                  src2_h, dst2_h, w2_h)

        plsc.subcore_barrier()
        _write_out(rows_a, acc_sh, o_h.at[c], s)

    return k(src2, dst2, w2, t)


def _tc_first(x, wcat):
    def body(x_ref, w_ref, o0_ref, o1_ref):
        u = jnp.dot(x_ref[...], w_ref[...],
                    preferred_element_type=jnp.float32)
        o0_ref[...] = u[:, :_H]
        o1_ref[...] = u[:, _H:]

    return pl.pallas_call(
        body,
        grid=(_NP // _ROWBLK,),
        in_specs=[pl.BlockSpec((_ROWBLK, _D), lambda i: (i, 0)),
                  pl.BlockSpec((_D, _D), lambda i: (0, 0))],
        out_specs=[pl.BlockSpec((_ROWBLK, _H), lambda i: (i, 0)),
                   pl.BlockSpec((_ROWBLK, _H), lambda i: (i, 0))],
        out_shape=[jax.ShapeDtypeStruct((_NP, _H), jnp.float32),
                   jax.ShapeDtypeStruct((_NP, _H), jnp.float32)],
    )(x, wcat)


def _tc_layer(keep, p0, p1, wcat, bvec):
    def body(k_ref, p0_ref, p1_ref, w_ref, b_ref, o0_ref, o1_ref):
        wl = w_ref[...]
        upper = p0_ref[...] + p1_ref[...]
        u = (jnp.dot(k_ref[...], wl[:_H, :],
                     preferred_element_type=jnp.float32)
             + jnp.dot(upper, wl[_H:, :],
                       preferred_element_type=jnp.float32)
             + jnp.dot(b_ref[...], wl, preferred_element_type=jnp.float32))
        o0_ref[...] = u[:, :_H]
        o1_ref[...] = u[:, _H:]

    return pl.pallas_call(
        body,
        grid=(_NP // _ROWBLK,),
        in_specs=[pl.BlockSpec((_ROWBLK, _H), lambda i: (i, 0)),
                  pl.BlockSpec((_ROWBLK, _H), lambda i: (i, 0)),
                  pl.BlockSpec((_ROWBLK, _H), lambda i: (i, 0)),
                  pl.BlockSpec((_D, _D), lambda i: (0, 0)),
                  pl.BlockSpec((1, _D), lambda i: (0, 0))],
        out_specs=[pl.BlockSpec((_ROWBLK, _H), lambda i: (i, 0)),
                   pl.BlockSpec((_ROWBLK, _H), lambda i: (i, 0))],
        out_shape=[jax.ShapeDtypeStruct((_NP, _H), jnp.float32),
                   jax.ShapeDtypeStruct((_NP, _H), jnp.float32)],
    )(keep, p0, p1, wcat, bvec)


def _tc_final(keep, p0, p1, wcp, bvec, bcp):
    def body(k_ref, p0_ref, p1_ref, w_ref, b_ref, bc_ref, o_ref):
        wl = w_ref[...]
        upper = p0_ref[...] + p1_ref[...]
        o_ref[...] = (jnp.dot(k_ref[...], wl[:_H, :],
                              preferred_element_type=jnp.float32)
                      + jnp.dot(upper, wl[_H:, :],
                                preferred_element_type=jnp.float32)
                      + jnp.dot(b_ref[...], wl,
                                preferred_element_type=jnp.float32)
                      + bc_ref[...])

    return pl.pallas_call(
        body,
        grid=(_NP // _ROWBLK,),
        in_specs=[pl.BlockSpec((_ROWBLK, _H), lambda i: (i, 0)),
                  pl.BlockSpec((_ROWBLK, _H), lambda i: (i, 0)),
                  pl.BlockSpec((_ROWBLK, _H), lambda i: (i, 0)),
                  pl.BlockSpec((_D, _D), lambda i: (0, 0)),
                  pl.BlockSpec((1, _D), lambda i: (0, 0)),
                  pl.BlockSpec((1, _D), lambda i: (0, 0))],
        out_specs=pl.BlockSpec((_ROWBLK, _D), lambda i: (i, 0)),
        out_shape=jax.ShapeDtypeStruct((_NP, _D), jnp.float32),
    )(keep, p0, p1, wcp, bvec, bcp)


def kernel(x, edge_index, edge_weight, W, b, Wc, bc):
    pad = _EP - _E
    src2 = jnp.concatenate(
        [edge_index[0], jnp.zeros((pad,), jnp.int32)]).reshape(_NCHUNK, _CH)
    dst2 = jnp.concatenate(
        [edge_index[1], jnp.zeros((pad,), jnp.int32)]).reshape(_NCHUNK, _CH)
    w2 = jnp.concatenate(
        [edge_weight, jnp.zeros((pad,), jnp.float32)]).reshape(_NCHUNK, _CH)
    xp = jnp.pad(x, ((0, _NP - _N), (0, 0)))
    nclass = Wc.shape[1]

    t0, t1 = _tc_first(xp, jnp.concatenate([W[0, 0], W[0, 1]], axis=1))
    for l in range(W.shape[0]):
        keep, upper = _spmm_hop1(src2, dst2, w2, t0, t1)
        parts = _spmm_hop2(src2, dst2, w2, upper)
        p0, p1 = parts[0], parts[1]
        bvec = jnp.concatenate([b[l, 0], b[l, 1]])[None, :]
        if l + 1 < W.shape[0]:
            wcat = jnp.concatenate([W[l + 1, 0], W[l + 1, 1]], axis=1)
            t0, t1 = _tc_layer(keep, p0, p1, wcat, bvec)
        else:
            wcp = jnp.pad(Wc, ((0, 0), (0, _D - nclass)))
            bcp = jnp.pad(bc, (0, _D - nclass))[None, :]
            out = _tc_final(keep, p0, p1, wcp, bvec, bcp)
            return out[:_N, :nclass]
